# baseline (speedup 1.0000x reference)
import functools

import jax
import jax.numpy as jnp
from jax import lax
from jax.experimental import pallas as pl
from jax.experimental.pallas import tpu as pltpu
from jax.experimental.pallas import tpu_sc as plsc

D_MODEL = 64
SCALE = 8.0
NC = 2
NS = 16
NW = NC * NS
N_ROWS = 4096 * 200
ROWS_PER_W = N_ROWS // NW
CHUNK = 512
G = 128
NG = CHUNK // G
NCHUNK = ROWS_PER_W // CHUNK

_mesh = plsc.VectorSubcoreMesh(core_axis_name="c", subcore_axis_name="s")


@functools.partial(
    pl.kernel,
    mesh=_mesh,
    out_type=jax.ShapeDtypeStruct((N_ROWS, D_MODEL), jnp.float32),
    scratch_types=[
        pltpu.VMEM((CHUNK,), jnp.int32),
        pltpu.VMEM((CHUNK, D_MODEL), jnp.float32),
        pltpu.SemaphoreType.DMA,
    ],
    compiler_params=pltpu.CompilerParams(use_tc_tiling_on_sc=False),
)
def _emb_lookup(x_hbm, lut_hbm, out_hbm, idx_v, rows_v, sem):
    wid = lax.axis_index("s") * NC + lax.axis_index("c")
    wbase = wid * ROWS_PER_W

    def chunk_body(s, carry):
        base = wbase + s * CHUNK
        pltpu.sync_copy(x_hbm.at[pl.ds(base, CHUNK)], idx_v)
        copies = [
            pltpu.async_copy(
                lut_hbm.at[idx_v.at[pl.ds(j * G, G)]],
                rows_v.at[pl.ds(j * G, G)],
                sem,
            )
            for j in range(NG)
        ]
        for c in copies:
            c.wait()

        def scale_body(i, c2):
            for j in range(D_MODEL // 16):
                sl = pl.ds(j * 16, 16)
                rows_v[i, sl] = rows_v[i, sl] * SCALE
            return c2

        lax.fori_loop(0, CHUNK, scale_body, 0)
        pltpu.sync_copy(rows_v, out_hbm.at[pl.ds(base, CHUNK)])
        return carry

    lax.fori_loop(0, NCHUNK, chunk_body, 0)


def kernel(x, lut):
    out = _emb_lookup(x.reshape(N_ROWS), lut)
    return out.reshape(x.shape[0], x.shape[1], D_MODEL)


# speedup vs baseline: 1.1316x; 1.1316x over previous
import functools

import jax
import jax.numpy as jnp
from jax import lax
from jax.experimental import pallas as pl
from jax.experimental.pallas import tpu as pltpu
from jax.experimental.pallas import tpu_sc as plsc

D_MODEL = 64
SCALE = 8.0
NC = 2
NS = 16
NW = NC * NS
N_ROWS = 4096 * 200
ROWS_PER_W = N_ROWS // NW
CHUNK = 512
G = 128
NG = CHUNK // G
NCHUNK = ROWS_PER_W // CHUNK
NBUF = 3

_mesh = plsc.VectorSubcoreMesh(core_axis_name="c", subcore_axis_name="s")


@functools.partial(
    pl.kernel,
    mesh=_mesh,
    out_type=jax.ShapeDtypeStruct((N_ROWS, D_MODEL), jnp.float32),
    scratch_types=[
        pltpu.VMEM((ROWS_PER_W,), jnp.int32),
        pltpu.VMEM((NBUF, CHUNK, D_MODEL), jnp.float32),
        pltpu.SemaphoreType.DMA,
        pltpu.SemaphoreType.DMA,
    ],
    compiler_params=pltpu.CompilerParams(use_tc_tiling_on_sc=False),
)
def _emb_lookup(x_hbm, lut_hbm, out_hbm, idx_v, rows_v, sem_g, sem_o):
    wid = lax.axis_index("s") * NC + lax.axis_index("c")
    wbase = wid * ROWS_PER_W

    pltpu.sync_copy(x_hbm.at[pl.ds(wbase, ROWS_PER_W)], idx_v)

    def fire_gather(s, slot):
        for j in range(NG):
            pltpu.async_copy(
                lut_hbm.at[idx_v.at[pl.ds(s * CHUNK + j * G, G)]],
                rows_v.at[slot, pl.ds(j * G, G)],
                sem_g,
            )

    def wait_gather():
        pltpu.make_async_copy(
            lut_hbm.at[pl.ds(0, CHUNK)], rows_v.at[0], sem_g
        ).wait()

    def fire_out(s, slot):
        pltpu.async_copy(
            rows_v.at[slot],
            out_hbm.at[pl.ds(wbase + s * CHUNK, CHUNK)],
            sem_o,
        )

    def wait_out():
        pltpu.make_async_copy(
            rows_v.at[0], out_hbm.at[pl.ds(wbase, CHUNK)], sem_o
        ).wait()

    def scale(slot):
        def scale_body(i, c):
            for j in range(D_MODEL // 16):
                sl = pl.ds(j * 16, 16)
                rows_v[slot, i, sl] = rows_v[slot, i, sl] * SCALE
            return c

        lax.fori_loop(0, CHUNK, scale_body, 0, unroll=4)

    def step(t, slot, gslot, fire, guard_out_wait):
        wait_gather()
        if guard_out_wait:
            @pl.when(t >= 1)
            def _():
                wait_out()
        else:
            wait_out()
        if fire:
            fire_gather(t + 2, gslot)
        scale(slot)
        fire_out(t, slot)

    fire_gather(0, 0)
    fire_gather(1, 1)

    def tri_body(i, c):
        t0 = 3 * i
        for r in range(3):
            step(t0 + r, r, (r + 2) % 3, fire=True, guard_out_wait=(r == 0))
        return c

    lax.fori_loop(0, (NCHUNK - 2) // 3, tri_body, 0)
    step(NCHUNK - 2, (NCHUNK - 2) % NBUF, 0, fire=False, guard_out_wait=False)
    step(NCHUNK - 1, (NCHUNK - 1) % NBUF, 0, fire=False, guard_out_wait=False)
    wait_out()


def kernel(x, lut):
    out = _emb_lookup(x.reshape(N_ROWS), lut)
    return out.reshape(x.shape[0], x.shape[1], D_MODEL)
